# baseline (speedup 1.0000x reference)
import functools

import jax
import jax.numpy as jnp
import numpy as np
from jax import lax
from jax.experimental import pallas as pl
from jax.experimental.pallas import tpu as pltpu
from jax.experimental.pallas import tpu_sc as plsc

B = 4
A = 9
H = 50
W = 50
N = A * H * W
NP = 22528
ROWS = NP // 128
M = 6144
PRE_NMS_TOPN = 6000
POST_NMS_TOPN = 1000
OUT_SLOTS = 1024
MIN_THRESHOLD = 16.0
NMS_THRESH = 0.7
IMG_W = 800.0
IMG_H = 800.0
NEG = -1.0e30


def _anchor_consts():
    base_size, ratios, scales, stride = 16, np.array([0.5, 1.0, 2.0]), np.array([8.0, 16.0, 32.0]), 16

    def whctrs(a):
        w = a[2] - a[0] + 1.0
        h = a[3] - a[1] + 1.0
        return w, h, a[0] + 0.5 * (w - 1.0), a[1] + 0.5 * (h - 1.0)

    def mk(ws, hs, xc, yc):
        ws = ws[:, None]
        hs = hs[:, None]
        return np.hstack([xc - 0.5 * (ws - 1.0), yc - 0.5 * (hs - 1.0),
                          xc + 0.5 * (ws - 1.0), yc + 0.5 * (hs - 1.0)])

    basea = np.array([0.0, 0.0, base_size - 1.0, base_size - 1.0])
    w, h, xc, yc = whctrs(basea)
    size = w * h
    ws = np.round(np.sqrt(size / ratios))
    hs = np.round(ws * ratios)
    ra = mk(ws, hs, xc, yc)
    out = []
    for i in range(ra.shape[0]):
        w, h, xc, yc = whctrs(ra[i])
        out.append(mk(w * scales, h * scales, xc, yc))
    anchors = np.vstack(out).astype(np.float32)

    sx = np.arange(W) * stride
    sy = np.arange(H) * stride
    mx, my = np.meshgrid(sx, sy)
    shifts = np.stack([mx.ravel(), my.ravel(), mx.ravel(), my.ravel()], axis=1)
    shifts = shifts.reshape(-1, 1, 4).astype(np.float32)
    alla = (anchors + shifts).reshape(-1, 4)

    aw = alla[:, 2] - alla[:, 0] + 1.0
    ah = alla[:, 3] - alla[:, 1] + 1.0
    ax = alla[:, 0] + 0.5 * aw
    ay = alla[:, 1] + 0.5 * ah

    def padr(v, fill):
        return np.concatenate([v, np.full((NP - N,), fill, np.float32)]).reshape(ROWS, 128)

    return (padr(aw, 1.0), padr(ah, 1.0), padr(ax, 0.0), padr(ay, 0.0))


_AW, _AH, _AX, _AY = _anchor_consts()


def _k1_body(dx, dy, dw, dh, sc, aw, ah, ax, ay,
             x1o, y1o, x2o, y2o, mso, kco):
    awv, ahv, axv, ayv = aw[...], ah[...], ax[...], ay[...]
    pcx = dx[...] * awv + axv
    pcy = dy[...] * ahv + ayv
    pw = jnp.exp(dw[...]) * awv
    ph = jnp.exp(dh[...]) * ahv
    x1 = jnp.clip(pcx - 0.5 * pw, 0.0, IMG_W - 1.0)
    y1 = jnp.clip(pcy - 0.5 * ph, 0.0, IMG_H - 1.0)
    x2 = jnp.clip(pcx + 0.5 * pw, 0.0, IMG_W - 1.0)
    y2 = jnp.clip(pcy + 0.5 * ph, 0.0, IMG_H - 1.0)
    keep = ((x2 - x1 + 1.0 >= MIN_THRESHOLD)
            & (y2 - y1 + 1.0 >= MIN_THRESHOLD))
    x1o[...] = x1
    y1o[...] = y1
    x2o[...] = x2
    y2o[...] = y2
    mso[...] = jnp.where(keep, sc[...], NEG)
    kc = jnp.sum(keep.astype(jnp.float32), axis=(1, 2), keepdims=True)
    kco[...] = jnp.broadcast_to(kc, (B, 8, 128))


def _run_k1(dx, dy, dw, dh, sc):
    outs = pl.pallas_call(
        _k1_body,
        out_shape=(
            jax.ShapeDtypeStruct((B, ROWS, 128), jnp.float32),
            jax.ShapeDtypeStruct((B, ROWS, 128), jnp.float32),
            jax.ShapeDtypeStruct((B, ROWS, 128), jnp.float32),
            jax.ShapeDtypeStruct((B, ROWS, 128), jnp.float32),
            jax.ShapeDtypeStruct((B, ROWS, 128), jnp.float32),
            jax.ShapeDtypeStruct((B, 8, 128), jnp.float32),
        ),
    )(dx, dy, dw, dh, sc,
      jnp.asarray(_AW), jnp.asarray(_AH), jnp.asarray(_AX), jnp.asarray(_AY))
    return outs


_TI = 512
_CJ = 2048


def _k2_body(msi, msj, ranko):
    t = pl.program_id(1)
    si = msi[0]
    ii = t * _TI + lax.broadcasted_iota(jnp.int32, (_TI, 1), 0)
    acc = jnp.zeros((_TI, 1), jnp.float32)
    for c in range(NP // _CJ):
        sj = msj[0, :, c * _CJ:(c + 1) * _CJ]
        jj = c * _CJ + lax.broadcasted_iota(jnp.int32, (1, _CJ), 1)
        gt = sj > si
        tie = (sj == si) & (jj < ii)
        acc = acc + jnp.sum((gt | tie).astype(jnp.float32), axis=1,
                            keepdims=True)
    ranko[0] = acc.astype(jnp.int32)


def _run_k2(ms):
    msi = ms.reshape(B, NP, 1)
    msj = ms.reshape(B, 1, NP)
    rank = pl.pallas_call(
        _k2_body,
        grid=(B, NP // _TI),
        in_specs=[
            pl.BlockSpec((1, _TI, 1), lambda b, t: (b, t, 0)),
            pl.BlockSpec((1, 1, NP), lambda b, t: (b, 0, 0)),
        ],
        out_specs=pl.BlockSpec((1, _TI, 1), lambda b, t: (b, t, 0)),
        out_shape=jax.ShapeDtypeStruct((B, NP, 1), jnp.int32),
    )(msi, msj)
    return rank.reshape(B, NP)


_MC = NP // 2
_NC = 2
_NS = 16


def _k3_body(rank_h, x1_h, y1_h, x2_h, y2_h, ss_h,
             o1_h, o2_h, o3_h, o4_h, o5_h,
             rank_v, i1, i2, i3, i4, i5, o1, o2, o3, o4, o5):
    wid = lax.axis_index("c") * _NS + lax.axis_index("s")

    @pl.when(wid < B)
    def _():
        bidx = wid

        def macro(mc, carry):
            off = pl.multiple_of(mc * _MC, _MC)
            pltpu.sync_copy(rank_h.at[bidx, pl.ds(off, _MC)], rank_v)
            pltpu.sync_copy(x1_h.at[bidx, pl.ds(off, _MC)], i1)
            pltpu.sync_copy(y1_h.at[bidx, pl.ds(off, _MC)], i2)
            pltpu.sync_copy(x2_h.at[bidx, pl.ds(off, _MC)], i3)
            pltpu.sync_copy(y2_h.at[bidx, pl.ds(off, _MC)], i4)
            pltpu.sync_copy(ss_h.at[bidx, pl.ds(off, _MC)], i5)

            def chunk(k, c2):
                o16 = pl.multiple_of(k * 16, 16)
                idx = rank_v[pl.ds(o16, 16)]
                msk = idx < M
                idxc = jnp.where(msk, idx, 0)
                plsc.store_scatter(o1, [idxc], i1[pl.ds(o16, 16)], mask=msk)
                plsc.store_scatter(o2, [idxc], i2[pl.ds(o16, 16)], mask=msk)
                plsc.store_scatter(o3, [idxc], i3[pl.ds(o16, 16)], mask=msk)
                plsc.store_scatter(o4, [idxc], i4[pl.ds(o16, 16)], mask=msk)
                plsc.store_scatter(o5, [idxc], i5[pl.ds(o16, 16)], mask=msk)
                return c2

            lax.fori_loop(0, _MC // 16, chunk, 0)
            return carry

        lax.fori_loop(0, NP // _MC, macro, 0)
        pltpu.sync_copy(o1, o1_h.at[bidx])
        pltpu.sync_copy(o2, o2_h.at[bidx])
        pltpu.sync_copy(o3, o3_h.at[bidx])
        pltpu.sync_copy(o4, o4_h.at[bidx])
        pltpu.sync_copy(o5, o5_h.at[bidx])


def _run_k3(rank, x1, y1, x2, y2, ms):
    fvec = jax.ShapeDtypeStruct((B, M), jnp.float32)
    run = pl.kernel(
        _k3_body,
        out_type=(fvec,) * 5,
        compiler_params=pltpu.CompilerParams(needs_layout_passes=False),
        mesh=plsc.VectorSubcoreMesh(core_axis_name="c", subcore_axis_name="s"),
        scratch_types=[
            pltpu.VMEM((_MC,), jnp.int32),
            pltpu.VMEM((_MC,), jnp.float32),
            pltpu.VMEM((_MC,), jnp.float32),
            pltpu.VMEM((_MC,), jnp.float32),
            pltpu.VMEM((_MC,), jnp.float32),
            pltpu.VMEM((_MC,), jnp.float32),
            pltpu.VMEM((M,), jnp.float32),
            pltpu.VMEM((M,), jnp.float32),
            pltpu.VMEM((M,), jnp.float32),
            pltpu.VMEM((M,), jnp.float32),
            pltpu.VMEM((M,), jnp.float32),
        ],
    )
    return run(rank, x1, y1, x2, y2, ms)


_MR = M // 128
_CHR = 8


def _k4_body(x1r, y1r, x2r, y2r, kcr, kepto):
    kc = jnp.min(kcr[...]).astype(jnp.int32)
    pre_n = jnp.minimum(PRE_NMS_TOPN, kc)
    sub = lax.broadcasted_iota(jnp.int32, (B, _MR, 128), 1)
    lan = lax.broadcasted_iota(jnp.int32, (B, _MR, 128), 2)
    pos = sub * 128 + lan
    active = (pos < pre_n).astype(jnp.float32)
    lanes = lax.broadcasted_iota(jnp.int32, (B, 1, 128), 2)
    kept_rows = []
    x1f, y1f, x2f, y2f = x1r[...], y1r[...], x2r[...], y2r[...]

    def extract(row_arrs, msk):
        return [jnp.sum(a * msk, axis=(1, 2), keepdims=True) for a in row_arrs]

    for r in range(_MR):
        rx1 = x1f[:, r:r + 1, :]
        ry1 = y1f[:, r:r + 1, :]
        rx2 = x2f[:, r:r + 1, :]
        ry2 = y2f[:, r:r + 1, :]
        rar = (rx2 - rx1 + 1.0) * (ry2 - ry1 + 1.0)

        def inrow(jj, racts):
            ract = racts
            mjf = (lanes == jj).astype(jnp.float32)
            kb, ex1, ey1, ex2, ey2 = extract(
                (ract, rx1, ry1, rx2, ry2), mjf)
            aj = (ex2 - ex1 + 1.0) * (ey2 - ey1 + 1.0)
            xx1 = jnp.maximum(ex1, rx1)
            yy1 = jnp.maximum(ey1, ry1)
            xx2 = jnp.minimum(ex2, rx2)
            yy2 = jnp.minimum(ey2, ry2)
            ww = jnp.maximum(0.0, xx2 - xx1 + 1.0)
            hh = jnp.maximum(0.0, yy2 - yy1 + 1.0)
            inter = ww * hh
            iou = inter / (aj + rar - inter)
            supf = ((kb > 0.0) & (iou > NMS_THRESH)
                    & (lanes > jj)).astype(jnp.float32)
            return ract * (1.0 - supf)

        ract = lax.fori_loop(0, 128, inrow, active[:, :1, :])
        kept_rows.append(ract)

        if r + 1 < _MR:
            tx1 = x1f[:, r + 1:, :]
            ty1 = y1f[:, r + 1:, :]
            tx2 = x2f[:, r + 1:, :]
            ty2 = y2f[:, r + 1:, :]
            tar = (tx2 - tx1 + 1.0) * (ty2 - ty1 + 1.0)

            def cross(p, tact):
                mpf = (lanes == p).astype(jnp.float32)
                kb, ex1, ey1, ex2, ey2 = extract(
                    (ract, rx1, ry1, rx2, ry2), mpf)
                aj = (ex2 - ex1 + 1.0) * (ey2 - ey1 + 1.0)
                xx1 = jnp.maximum(ex1, tx1)
                yy1 = jnp.maximum(ey1, ty1)
                xx2 = jnp.minimum(ex2, tx2)
                yy2 = jnp.minimum(ey2, ty2)
                ww = jnp.maximum(0.0, xx2 - xx1 + 1.0)
                hh = jnp.maximum(0.0, yy2 - yy1 + 1.0)
                inter = ww * hh
                iou = inter / (aj + tar - inter)
                supf = ((kb > 0.0) & (iou > NMS_THRESH)).astype(jnp.float32)
                return tact * (1.0 - supf)

            active = lax.fori_loop(0, 128, cross, active[:, 1:, :])
    kepto[...] = jnp.concatenate(kept_rows, axis=1)


def _run_k4(xs1, ys1, xs2, ys2, kc):
    shp = (B, _MR, 128)
    kept = pl.pallas_call(
        _k4_body,
        out_shape=jax.ShapeDtypeStruct(shp, jnp.float32),
    )(xs1.reshape(shp), ys1.reshape(shp), xs2.reshape(shp), ys2.reshape(shp),
      kc)
    return kept.reshape(B, M)


def _k5_body(kept_h, x1_h, y1_h, x2_h, y2_h, ss_h,
             ob_h, os_h,
             kept_v, i1, i2, i3, i4, i5, ob, os_):
    wid = lax.axis_index("c") * _NS + lax.axis_index("s")

    @pl.when(wid < B)
    def _():
        bidx = wid
        pltpu.sync_copy(kept_h.at[bidx], kept_v)
        pltpu.sync_copy(x1_h.at[bidx], i1)
        pltpu.sync_copy(y1_h.at[bidx], i2)
        pltpu.sync_copy(x2_h.at[bidx], i3)
        pltpu.sync_copy(y2_h.at[bidx], i4)
        pltpu.sync_copy(ss_h.at[bidx], i5)

        zf = jnp.zeros((16,), jnp.float32)

        def zb(k, c):
            ob[pl.ds(pl.multiple_of(k * 16, 16), 16)] = zf
            return c

        lax.fori_loop(0, (OUT_SLOTS * 4) // 16, zb, 0)

        def zs(k, c):
            os_[pl.ds(pl.multiple_of(k * 16, 16), 16)] = zf
            return c

        lax.fori_loop(0, OUT_SLOTS // 16, zs, 0)

        def chunk(k, base):
            o16 = pl.multiple_of(k * 16, 16)
            kv = kept_v[pl.ds(o16, 16)] > 0.5
            ki = kv.astype(jnp.int32)
            slot = base + plsc.cumsum(ki) - 1
            m2 = kv & (slot < POST_NMS_TOPN)
            slotc = jnp.where(m2, slot, 0)
            plsc.store_scatter(os_, [slotc], i5[pl.ds(o16, 16)], mask=m2)
            sb = slotc * 4
            plsc.store_scatter(ob, [sb], i1[pl.ds(o16, 16)], mask=m2)
            plsc.store_scatter(ob, [sb + 1], i2[pl.ds(o16, 16)], mask=m2)
            plsc.store_scatter(ob, [sb + 2], i3[pl.ds(o16, 16)], mask=m2)
            plsc.store_scatter(ob, [sb + 3], i4[pl.ds(o16, 16)], mask=m2)
            return base + jnp.sum(ki)

        lax.fori_loop(0, M // 16, chunk, jnp.int32(0))
        pltpu.sync_copy(ob, ob_h.at[bidx])
        pltpu.sync_copy(os_, os_h.at[bidx])


def _run_k5(kept, xs1, ys1, xs2, ys2, ss):
    run = pl.kernel(
        _k5_body,
        out_type=(
            jax.ShapeDtypeStruct((B, OUT_SLOTS * 4), jnp.float32),
            jax.ShapeDtypeStruct((B, OUT_SLOTS), jnp.float32),
        ),
        compiler_params=pltpu.CompilerParams(needs_layout_passes=False),
        mesh=plsc.VectorSubcoreMesh(core_axis_name="c", subcore_axis_name="s"),
        scratch_types=[
            pltpu.VMEM((M,), jnp.float32),
            pltpu.VMEM((M,), jnp.float32),
            pltpu.VMEM((M,), jnp.float32),
            pltpu.VMEM((M,), jnp.float32),
            pltpu.VMEM((M,), jnp.float32),
            pltpu.VMEM((M,), jnp.float32),
            pltpu.VMEM((OUT_SLOTS * 4,), jnp.float32),
            pltpu.VMEM((OUT_SLOTS,), jnp.float32),
        ],
    )
    return run(kept, xs1, ys1, xs2, ys2, ss)


@jax.jit
def kernel(labels_pred, bbox_reg):
    scores = jnp.transpose(labels_pred, (0, 2, 3, 1)).reshape(B, N, 2)[..., 1]
    breg = jnp.transpose(bbox_reg, (0, 2, 3, 1)).reshape(B, N, 4)
    pad = ((0, 0), (0, NP - N))

    def prep(v):
        return jnp.pad(v, pad).reshape(B, ROWS, 128)

    dx = prep(breg[..., 0])
    dy = prep(breg[..., 1])
    dw = prep(breg[..., 2])
    dh = prep(breg[..., 3])
    sc = prep(scores)

    x1, y1, x2, y2, ms, kc = _run_k1(dx, dy, dw, dh, sc)
    flat = lambda v: v.reshape(B, NP)
    rank = _run_k2(flat(ms))
    xs1, ys1, xs2, ys2, ss = _run_k3(rank, flat(x1), flat(y1), flat(x2),
                                     flat(y2), flat(ms))
    kept = _run_k4(xs1, ys1, xs2, ys2, kc)
    ob, os_ = _run_k5(kept, xs1, ys1, xs2, ys2, ss)
    boxes_out = ob.reshape(B, OUT_SLOTS, 4)[:, :POST_NMS_TOPN]
    scores_out = os_[:, :POST_NMS_TOPN]
    return boxes_out, scores_out


# speedup vs baseline: 2.2916x; 2.2916x over previous
import functools

import jax
import jax.numpy as jnp
import numpy as np
from jax import lax
from jax.experimental import pallas as pl
from jax.experimental.pallas import tpu as pltpu
from jax.experimental.pallas import tpu_sc as plsc

B = 4
A = 9
H = 50
W = 50
N = A * H * W
NP = 22528
ROWS = NP // 128
M = 6144
PRE_NMS_TOPN = 6000
POST_NMS_TOPN = 1000
OUT_SLOTS = 1024
MIN_THRESHOLD = 16.0
NMS_THRESH = 0.7
IMG_W = 800.0
IMG_H = 800.0
NEG = -1.0e30


def _anchor_consts():
    base_size, ratios, scales, stride = 16, np.array([0.5, 1.0, 2.0]), np.array([8.0, 16.0, 32.0]), 16

    def whctrs(a):
        w = a[2] - a[0] + 1.0
        h = a[3] - a[1] + 1.0
        return w, h, a[0] + 0.5 * (w - 1.0), a[1] + 0.5 * (h - 1.0)

    def mk(ws, hs, xc, yc):
        ws = ws[:, None]
        hs = hs[:, None]
        return np.hstack([xc - 0.5 * (ws - 1.0), yc - 0.5 * (hs - 1.0),
                          xc + 0.5 * (ws - 1.0), yc + 0.5 * (hs - 1.0)])

    basea = np.array([0.0, 0.0, base_size - 1.0, base_size - 1.0])
    w, h, xc, yc = whctrs(basea)
    size = w * h
    ws = np.round(np.sqrt(size / ratios))
    hs = np.round(ws * ratios)
    ra = mk(ws, hs, xc, yc)
    out = []
    for i in range(ra.shape[0]):
        w, h, xc, yc = whctrs(ra[i])
        out.append(mk(w * scales, h * scales, xc, yc))
    anchors = np.vstack(out).astype(np.float32)

    sx = np.arange(W) * stride
    sy = np.arange(H) * stride
    mx, my = np.meshgrid(sx, sy)
    shifts = np.stack([mx.ravel(), my.ravel(), mx.ravel(), my.ravel()], axis=1)
    shifts = shifts.reshape(-1, 1, 4).astype(np.float32)
    alla = (anchors + shifts).reshape(-1, 4)

    aw = alla[:, 2] - alla[:, 0] + 1.0
    ah = alla[:, 3] - alla[:, 1] + 1.0
    ax = alla[:, 0] + 0.5 * aw
    ay = alla[:, 1] + 0.5 * ah

    def padr(v, fill):
        return np.concatenate([v, np.full((NP - N,), fill, np.float32)]).reshape(ROWS, 128)

    return (padr(aw, 1.0), padr(ah, 1.0), padr(ax, 0.0), padr(ay, 0.0))


_AW, _AH, _AX, _AY = _anchor_consts()


def _k1_body(dx, dy, dw, dh, sc, aw, ah, ax, ay,
             x1o, y1o, x2o, y2o, mso, kco):
    awv, ahv, axv, ayv = aw[...], ah[...], ax[...], ay[...]
    pcx = dx[...] * awv + axv
    pcy = dy[...] * ahv + ayv
    pw = jnp.exp(dw[...]) * awv
    ph = jnp.exp(dh[...]) * ahv
    x1 = jnp.clip(pcx - 0.5 * pw, 0.0, IMG_W - 1.0)
    y1 = jnp.clip(pcy - 0.5 * ph, 0.0, IMG_H - 1.0)
    x2 = jnp.clip(pcx + 0.5 * pw, 0.0, IMG_W - 1.0)
    y2 = jnp.clip(pcy + 0.5 * ph, 0.0, IMG_H - 1.0)
    keep = ((x2 - x1 + 1.0 >= MIN_THRESHOLD)
            & (y2 - y1 + 1.0 >= MIN_THRESHOLD))
    x1o[...] = x1
    y1o[...] = y1
    x2o[...] = x2
    y2o[...] = y2
    mso[...] = jnp.where(keep, sc[...], NEG)
    kc = jnp.sum(keep.astype(jnp.float32), axis=(1, 2), keepdims=True)
    kco[...] = jnp.broadcast_to(kc, (B, 8, 128))


def _run_k1(dx, dy, dw, dh, sc):
    outs = pl.pallas_call(
        _k1_body,
        out_shape=(
            jax.ShapeDtypeStruct((B, ROWS, 128), jnp.float32),
            jax.ShapeDtypeStruct((B, ROWS, 128), jnp.float32),
            jax.ShapeDtypeStruct((B, ROWS, 128), jnp.float32),
            jax.ShapeDtypeStruct((B, ROWS, 128), jnp.float32),
            jax.ShapeDtypeStruct((B, ROWS, 128), jnp.float32),
            jax.ShapeDtypeStruct((B, 8, 128), jnp.float32),
        ),
    )(dx, dy, dw, dh, sc,
      jnp.asarray(_AW), jnp.asarray(_AH), jnp.asarray(_AX), jnp.asarray(_AY))
    return outs


_NB = 1024


def _k2a_body(msj, kcr, flago):
    bi = pl.program_id(0)
    kc = kcr[0, 0, 0].astype(jnp.int32)
    tgt = jnp.minimum(jnp.int32(M), kc)
    s = msj[0]
    k = jax.lax.bitcast_convert_type(s, jnp.int32)
    m = jnp.where(k >= 0, k, k ^ jnp.int32(0x7FFFFFFF))
    kel = s > (NEG * 0.5)
    b1 = (m >> 22) + _NB // 2
    b2 = (m >> 12) & (_NB - 1)
    bcol = lax.broadcasted_iota(jnp.int32, (_NB, 1), 0)
    cab = jnp.zeros((_NB, 1), jnp.float32)
    hb = jnp.zeros((_NB, 1), jnp.float32)
    for c in range(NP // _CJ):
        b1c = b1[:, c * _CJ:(c + 1) * _CJ]
        kc_ = kel[:, c * _CJ:(c + 1) * _CJ]
        cab = cab + jnp.sum(((b1c > bcol) & kc_).astype(jnp.float32),
                            axis=1, keepdims=True)
        hb = hb + jnp.sum(((b1c == bcol) & kc_).astype(jnp.float32),
                          axis=1, keepdims=True)
    tgtf = tgt.astype(jnp.float32)
    t1 = jnp.sum((cab >= tgtf).astype(jnp.int32))
    cab_t1 = jnp.sum(jnp.where(bcol == t1, cab, 0.0))
    t2g = tgtf - cab_t1
    cab2 = jnp.zeros((_NB, 1), jnp.float32)
    for c in range(NP // _CJ):
        b1c = b1[:, c * _CJ:(c + 1) * _CJ]
        b2c = b2[:, c * _CJ:(c + 1) * _CJ]
        kc_ = kel[:, c * _CJ:(c + 1) * _CJ]
        cab2 = cab2 + jnp.sum(((b1c == t1) & (b2c > bcol)
                               & kc_).astype(jnp.float32),
                              axis=1, keepdims=True)
    t2 = jnp.sum((cab2 >= t2g).astype(jnp.int32))
    flag = kel & ((b1 > t1) | ((b1 == t1) & (b2 >= t2)))
    flago[0] = flag.astype(jnp.float32)


def _run_k2a(ms, kc):
    msj = ms.reshape(B, 1, NP)
    flag = pl.pallas_call(
        _k2a_body,
        grid=(B,),
        in_specs=[
            pl.BlockSpec((1, 1, NP), lambda b: (b, 0, 0)),
            pl.BlockSpec((1, 8, 128), lambda b: (b, 0, 0)),
        ],
        out_specs=pl.BlockSpec((1, 1, NP), lambda b: (b, 0, 0)),
        out_shape=jax.ShapeDtypeStruct((B, 1, NP), jnp.float32),
    )(msj, kc)
    return flag.reshape(B, NP)


NCAP = 8192


def _k2c_body(flag_h, x1_h, y1_h, x2_h, y2_h, ss_h,
              c1_h, c2_h, c3_h, c4_h, c5_h,
              fv, i1, i2, i3, i4, i5, o1, o2, o3, o4, o5):
    wid = lax.axis_index("c") * _NS + lax.axis_index("s")

    @pl.when(wid < B)
    def _():
        bidx = wid
        neg = jnp.full((16,), -1.0e38, jnp.float32)
        zf = jnp.zeros((16,), jnp.float32)

        def init(k2, c):
            o16 = pl.multiple_of(k2 * 16, 16)
            o1[pl.ds(o16, 16)] = zf
            o2[pl.ds(o16, 16)] = zf
            o3[pl.ds(o16, 16)] = zf
            o4[pl.ds(o16, 16)] = zf
            o5[pl.ds(o16, 16)] = neg
            return c

        lax.fori_loop(0, NCAP // 16, init, 0)

        def macro(mc, base0):
            off = pl.multiple_of(mc * _MC, _MC)
            pltpu.sync_copy(flag_h.at[bidx, pl.ds(off, _MC)], fv)
            pltpu.sync_copy(x1_h.at[bidx, pl.ds(off, _MC)], i1)
            pltpu.sync_copy(y1_h.at[bidx, pl.ds(off, _MC)], i2)
            pltpu.sync_copy(x2_h.at[bidx, pl.ds(off, _MC)], i3)
            pltpu.sync_copy(y2_h.at[bidx, pl.ds(off, _MC)], i4)
            pltpu.sync_copy(ss_h.at[bidx, pl.ds(off, _MC)], i5)

            def chunk(k2, base):
                o16 = pl.multiple_of(k2 * 16, 16)
                kv = fv[pl.ds(o16, 16)] > 0.5
                ki = kv.astype(jnp.int32)
                slot = base + plsc.cumsum(ki) - 1
                m2 = kv & (slot < NCAP)
                slotc = jnp.where(m2, slot, 0)
                plsc.store_scatter(o1, [slotc], i1[pl.ds(o16, 16)], mask=m2)
                plsc.store_scatter(o2, [slotc], i2[pl.ds(o16, 16)], mask=m2)
                plsc.store_scatter(o3, [slotc], i3[pl.ds(o16, 16)], mask=m2)
                plsc.store_scatter(o4, [slotc], i4[pl.ds(o16, 16)], mask=m2)
                plsc.store_scatter(o5, [slotc], i5[pl.ds(o16, 16)], mask=m2)
                return base + jnp.sum(ki)

            return lax.fori_loop(0, _MC // 16, chunk, base0)

        lax.fori_loop(0, NP // _MC, macro, jnp.int32(0))
        pltpu.sync_copy(o1, c1_h.at[bidx])
        pltpu.sync_copy(o2, c2_h.at[bidx])
        pltpu.sync_copy(o3, c3_h.at[bidx])
        pltpu.sync_copy(o4, c4_h.at[bidx])
        pltpu.sync_copy(o5, c5_h.at[bidx])


def _run_k2c(flag, x1, y1, x2, y2, ms):
    fvec = jax.ShapeDtypeStruct((B, NCAP), jnp.float32)
    run = pl.kernel(
        _k2c_body,
        out_type=(fvec,) * 5,
        compiler_params=pltpu.CompilerParams(needs_layout_passes=False),
        mesh=plsc.VectorSubcoreMesh(core_axis_name="c", subcore_axis_name="s"),
        scratch_types=[pltpu.VMEM((_MC,), jnp.float32)] * 6
        + [pltpu.VMEM((NCAP,), jnp.float32)] * 5,
    )
    return run(flag, x1, y1, x2, y2, ms)


_TI = 512
_CJ = 2048


def _k2_body(msi, msj, ranko):
    t = pl.program_id(1)
    si = msi[0]
    ii = t * _TI + lax.broadcasted_iota(jnp.int32, (_TI, 1), 0)
    acc = jnp.zeros((_TI, 1), jnp.float32)
    for c in range(NCAP // _CJ):
        sj = msj[0, :, c * _CJ:(c + 1) * _CJ]
        jj = c * _CJ + lax.broadcasted_iota(jnp.int32, (1, _CJ), 1)
        gt = sj > si
        tie = (sj == si) & (jj < ii)
        acc = acc + jnp.sum((gt | tie).astype(jnp.float32), axis=1,
                            keepdims=True)
    ranko[0] = acc.astype(jnp.int32)


def _run_k2(cms):
    msi = cms.reshape(B, NCAP, 1)
    msj = cms.reshape(B, 1, NCAP)
    rank = pl.pallas_call(
        _k2_body,
        grid=(B, NCAP // _TI),
        in_specs=[
            pl.BlockSpec((1, _TI, 1), lambda b, t: (b, t, 0)),
            pl.BlockSpec((1, 1, NCAP), lambda b, t: (b, 0, 0)),
        ],
        out_specs=pl.BlockSpec((1, _TI, 1), lambda b, t: (b, t, 0)),
        out_shape=jax.ShapeDtypeStruct((B, NCAP, 1), jnp.int32),
    )(msi, msj)
    return rank.reshape(B, NCAP)


_MC = NP // 2
_NC = 2
_NS = 16


def _k3_body(rank_h, x1_h, y1_h, x2_h, y2_h, ss_h,
             o1_h, o2_h, o3_h, o4_h, o5_h,
             rank_v, i1, i2, i3, i4, i5, o1, o2, o3, o4, o5):
    wid = lax.axis_index("c") * _NS + lax.axis_index("s")

    @pl.when(wid < B)
    def _():
        bidx = wid
        pltpu.sync_copy(rank_h.at[bidx], rank_v)
        pltpu.sync_copy(x1_h.at[bidx], i1)
        pltpu.sync_copy(y1_h.at[bidx], i2)
        pltpu.sync_copy(x2_h.at[bidx], i3)
        pltpu.sync_copy(y2_h.at[bidx], i4)
        pltpu.sync_copy(ss_h.at[bidx], i5)

        def chunk(k, c2):
            o16 = pl.multiple_of(k * 16, 16)
            idx = rank_v[pl.ds(o16, 16)]
            msk = idx < M
            idxc = jnp.where(msk, idx, 0)
            plsc.store_scatter(o1, [idxc], i1[pl.ds(o16, 16)], mask=msk)
            plsc.store_scatter(o2, [idxc], i2[pl.ds(o16, 16)], mask=msk)
            plsc.store_scatter(o3, [idxc], i3[pl.ds(o16, 16)], mask=msk)
            plsc.store_scatter(o4, [idxc], i4[pl.ds(o16, 16)], mask=msk)
            plsc.store_scatter(o5, [idxc], i5[pl.ds(o16, 16)], mask=msk)
            return c2

        lax.fori_loop(0, NCAP // 16, chunk, 0)
        pltpu.sync_copy(o1, o1_h.at[bidx])
        pltpu.sync_copy(o2, o2_h.at[bidx])
        pltpu.sync_copy(o3, o3_h.at[bidx])
        pltpu.sync_copy(o4, o4_h.at[bidx])
        pltpu.sync_copy(o5, o5_h.at[bidx])


def _run_k3(rank, x1, y1, x2, y2, ms):
    fvec = jax.ShapeDtypeStruct((B, M), jnp.float32)
    run = pl.kernel(
        _k3_body,
        out_type=(fvec,) * 5,
        compiler_params=pltpu.CompilerParams(needs_layout_passes=False),
        mesh=plsc.VectorSubcoreMesh(core_axis_name="c", subcore_axis_name="s"),
        scratch_types=[
            pltpu.VMEM((NCAP,), jnp.int32),
            pltpu.VMEM((NCAP,), jnp.float32),
            pltpu.VMEM((NCAP,), jnp.float32),
            pltpu.VMEM((NCAP,), jnp.float32),
            pltpu.VMEM((NCAP,), jnp.float32),
            pltpu.VMEM((NCAP,), jnp.float32),
            pltpu.VMEM((M,), jnp.float32),
            pltpu.VMEM((M,), jnp.float32),
            pltpu.VMEM((M,), jnp.float32),
            pltpu.VMEM((M,), jnp.float32),
            pltpu.VMEM((M,), jnp.float32),
        ],
    )
    return run(rank, x1, y1, x2, y2, ms)


_MR = M // 128
_CHR = 8


def _k4_body(x1r, y1r, x2r, y2r, kcr, kepto):
    kc = jnp.min(kcr[...]).astype(jnp.int32)
    pre_n = jnp.minimum(PRE_NMS_TOPN, kc)
    sub = lax.broadcasted_iota(jnp.int32, (B, _MR, 128), 1)
    lan = lax.broadcasted_iota(jnp.int32, (B, _MR, 128), 2)
    pos = sub * 128 + lan
    active = (pos < pre_n).astype(jnp.float32)
    ploc = (lax.broadcasted_iota(jnp.int32, (B, _CHR, 128), 1) * 128
            + lax.broadcasted_iota(jnp.int32, (B, _CHR, 128), 2))
    kept_chunks = []
    x1f, y1f, x2f, y2f = x1r[...], y1r[...], x2r[...], y2r[...]
    for k in range(_MR // _CHR):
        r0 = k * _CHR
        x1 = x1f[:, r0:, :]
        y1 = y1f[:, r0:, :]
        x2 = x2f[:, r0:, :]
        y2 = y2f[:, r0:, :]
        areas = (x2 - x1 + 1.0) * (y2 - y1 + 1.0)
        prel = pos[:, r0:, :] - r0 * 128
        xc1 = x1[:, :_CHR, :]
        yc1 = y1[:, :_CHR, :]
        xc2 = x2[:, :_CHR, :]
        yc2 = y2[:, :_CHR, :]

        def body(jj, st):
            act, keptc = st
            mjf = (ploc == jj).astype(jnp.float32)
            actc = act[:, :_CHR, :]
            kb = jnp.sum(actc * mjf, axis=(1, 2), keepdims=True)
            ex1 = jnp.sum(xc1 * mjf, axis=(1, 2), keepdims=True)
            ey1 = jnp.sum(yc1 * mjf, axis=(1, 2), keepdims=True)
            ex2 = jnp.sum(xc2 * mjf, axis=(1, 2), keepdims=True)
            ey2 = jnp.sum(yc2 * mjf, axis=(1, 2), keepdims=True)
            aj = (ex2 - ex1 + 1.0) * (ey2 - ey1 + 1.0)
            xx1 = jnp.maximum(ex1, x1)
            yy1 = jnp.maximum(ey1, y1)
            xx2 = jnp.minimum(ex2, x2)
            yy2 = jnp.minimum(ey2, y2)
            ww = jnp.maximum(0.0, xx2 - xx1 + 1.0)
            hh = jnp.maximum(0.0, yy2 - yy1 + 1.0)
            inter = ww * hh
            iou = inter / (aj + areas - inter)
            supf = ((kb > 0.0) & (iou > NMS_THRESH)
                    & (prel > jj)).astype(jnp.float32)
            keptc = jnp.maximum(keptc, actc * mjf)
            act = act * (1.0 - supf)
            return act, keptc

        active, keptc = lax.fori_loop(
            0, _CHR * 128, body,
            (active, jnp.zeros((B, _CHR, 128), jnp.float32)))
        kept_chunks.append(keptc)
        active = active[:, _CHR:, :]
    kepto[...] = jnp.concatenate(kept_chunks, axis=1)


def _run_k4(xs1, ys1, xs2, ys2, kc):
    shp = (B, _MR, 128)
    kept = pl.pallas_call(
        _k4_body,
        out_shape=jax.ShapeDtypeStruct(shp, jnp.float32),
    )(xs1.reshape(shp), ys1.reshape(shp), xs2.reshape(shp), ys2.reshape(shp),
      kc)
    return kept.reshape(B, M)


def _k5_body(kept_h, x1_h, y1_h, x2_h, y2_h, ss_h,
             ob_h, os_h,
             kept_v, i1, i2, i3, i4, i5, ob, os_):
    wid = lax.axis_index("c") * _NS + lax.axis_index("s")

    @pl.when(wid < B)
    def _():
        bidx = wid
        pltpu.sync_copy(kept_h.at[bidx], kept_v)
        pltpu.sync_copy(x1_h.at[bidx], i1)
        pltpu.sync_copy(y1_h.at[bidx], i2)
        pltpu.sync_copy(x2_h.at[bidx], i3)
        pltpu.sync_copy(y2_h.at[bidx], i4)
        pltpu.sync_copy(ss_h.at[bidx], i5)

        zf = jnp.zeros((16,), jnp.float32)

        def zb(k, c):
            ob[pl.ds(pl.multiple_of(k * 16, 16), 16)] = zf
            return c

        lax.fori_loop(0, (OUT_SLOTS * 4) // 16, zb, 0)

        def zs(k, c):
            os_[pl.ds(pl.multiple_of(k * 16, 16), 16)] = zf
            return c

        lax.fori_loop(0, OUT_SLOTS // 16, zs, 0)

        def chunk(k, base):
            o16 = pl.multiple_of(k * 16, 16)
            kv = kept_v[pl.ds(o16, 16)] > 0.5
            ki = kv.astype(jnp.int32)
            slot = base + plsc.cumsum(ki) - 1
            m2 = kv & (slot < POST_NMS_TOPN)
            slotc = jnp.where(m2, slot, 0)
            plsc.store_scatter(os_, [slotc], i5[pl.ds(o16, 16)], mask=m2)
            sb = slotc * 4
            plsc.store_scatter(ob, [sb], i1[pl.ds(o16, 16)], mask=m2)
            plsc.store_scatter(ob, [sb + 1], i2[pl.ds(o16, 16)], mask=m2)
            plsc.store_scatter(ob, [sb + 2], i3[pl.ds(o16, 16)], mask=m2)
            plsc.store_scatter(ob, [sb + 3], i4[pl.ds(o16, 16)], mask=m2)
            return base + jnp.sum(ki)

        lax.fori_loop(0, M // 16, chunk, jnp.int32(0))
        pltpu.sync_copy(ob, ob_h.at[bidx])
        pltpu.sync_copy(os_, os_h.at[bidx])


def _run_k5(kept, xs1, ys1, xs2, ys2, ss):
    run = pl.kernel(
        _k5_body,
        out_type=(
            jax.ShapeDtypeStruct((B, OUT_SLOTS * 4), jnp.float32),
            jax.ShapeDtypeStruct((B, OUT_SLOTS), jnp.float32),
        ),
        compiler_params=pltpu.CompilerParams(needs_layout_passes=False),
        mesh=plsc.VectorSubcoreMesh(core_axis_name="c", subcore_axis_name="s"),
        scratch_types=[
            pltpu.VMEM((M,), jnp.float32),
            pltpu.VMEM((M,), jnp.float32),
            pltpu.VMEM((M,), jnp.float32),
            pltpu.VMEM((M,), jnp.float32),
            pltpu.VMEM((M,), jnp.float32),
            pltpu.VMEM((M,), jnp.float32),
            pltpu.VMEM((OUT_SLOTS * 4,), jnp.float32),
            pltpu.VMEM((OUT_SLOTS,), jnp.float32),
        ],
    )
    return run(kept, xs1, ys1, xs2, ys2, ss)


@jax.jit
def kernel(labels_pred, bbox_reg):
    scores = jnp.transpose(labels_pred, (0, 2, 3, 1)).reshape(B, N, 2)[..., 1]
    breg = jnp.transpose(bbox_reg, (0, 2, 3, 1)).reshape(B, N, 4)
    pad = ((0, 0), (0, NP - N))

    def prep(v):
        return jnp.pad(v, pad).reshape(B, ROWS, 128)

    dx = prep(breg[..., 0])
    dy = prep(breg[..., 1])
    dw = prep(breg[..., 2])
    dh = prep(breg[..., 3])
    sc = prep(scores)

    x1, y1, x2, y2, ms, kc = _run_k1(dx, dy, dw, dh, sc)
    flat = lambda v: v.reshape(B, NP)
    flag = _run_k2a(flat(ms), kc)
    c1, c2, c3, c4, c5 = _run_k2c(flag, flat(x1), flat(y1), flat(x2),
                                  flat(y2), flat(ms))
    crank = _run_k2(c5)
    xs1, ys1, xs2, ys2, ss = _run_k3(crank, c1, c2, c3, c4, c5)
    kept = _run_k4(xs1, ys1, xs2, ys2, kc)
    ob, os_ = _run_k5(kept, xs1, ys1, xs2, ys2, ss)
    boxes_out = ob.reshape(B, OUT_SLOTS, 4)[:, :POST_NMS_TOPN]
    scores_out = os_[:, :POST_NMS_TOPN]
    return boxes_out, scores_out


# speedup vs baseline: 3.5519x; 1.5500x over previous
import functools

import jax
import jax.numpy as jnp
import numpy as np
from jax import lax
from jax.experimental import pallas as pl
from jax.experimental.pallas import tpu as pltpu
from jax.experimental.pallas import tpu_sc as plsc

B = 4
A = 9
H = 50
W = 50
N = A * H * W
NP = 22528
ROWS = NP // 128
M = 6144
PRE_NMS_TOPN = 6000
POST_NMS_TOPN = 1000
OUT_SLOTS = 1024
MIN_THRESHOLD = 16.0
NMS_THRESH = 0.7
IMG_W = 800.0
IMG_H = 800.0
NEG = -1.0e30


def _anchor_consts():
    base_size, ratios, scales, stride = 16, np.array([0.5, 1.0, 2.0]), np.array([8.0, 16.0, 32.0]), 16

    def whctrs(a):
        w = a[2] - a[0] + 1.0
        h = a[3] - a[1] + 1.0
        return w, h, a[0] + 0.5 * (w - 1.0), a[1] + 0.5 * (h - 1.0)

    def mk(ws, hs, xc, yc):
        ws = ws[:, None]
        hs = hs[:, None]
        return np.hstack([xc - 0.5 * (ws - 1.0), yc - 0.5 * (hs - 1.0),
                          xc + 0.5 * (ws - 1.0), yc + 0.5 * (hs - 1.0)])

    basea = np.array([0.0, 0.0, base_size - 1.0, base_size - 1.0])
    w, h, xc, yc = whctrs(basea)
    size = w * h
    ws = np.round(np.sqrt(size / ratios))
    hs = np.round(ws * ratios)
    ra = mk(ws, hs, xc, yc)
    out = []
    for i in range(ra.shape[0]):
        w, h, xc, yc = whctrs(ra[i])
        out.append(mk(w * scales, h * scales, xc, yc))
    anchors = np.vstack(out).astype(np.float32)

    sx = np.arange(W) * stride
    sy = np.arange(H) * stride
    mx, my = np.meshgrid(sx, sy)
    shifts = np.stack([mx.ravel(), my.ravel(), mx.ravel(), my.ravel()], axis=1)
    shifts = shifts.reshape(-1, 1, 4).astype(np.float32)
    alla = (anchors + shifts).reshape(-1, 4)

    aw = alla[:, 2] - alla[:, 0] + 1.0
    ah = alla[:, 3] - alla[:, 1] + 1.0
    ax = alla[:, 0] + 0.5 * aw
    ay = alla[:, 1] + 0.5 * ah

    def padr(v, fill):
        return np.concatenate([v, np.full((NP - N,), fill, np.float32)]).reshape(ROWS, 128)

    return (padr(aw, 1.0), padr(ah, 1.0), padr(ax, 0.0), padr(ay, 0.0))


_AW, _AH, _AX, _AY = _anchor_consts()


def _k1_body(dx, dy, dw, dh, sc, aw, ah, ax, ay,
             x1o, y1o, x2o, y2o, mso, kco):
    awv, ahv, axv, ayv = aw[...], ah[...], ax[...], ay[...]
    pcx = dx[...] * awv + axv
    pcy = dy[...] * ahv + ayv
    pw = jnp.exp(dw[...]) * awv
    ph = jnp.exp(dh[...]) * ahv
    x1 = jnp.clip(pcx - 0.5 * pw, 0.0, IMG_W - 1.0)
    y1 = jnp.clip(pcy - 0.5 * ph, 0.0, IMG_H - 1.0)
    x2 = jnp.clip(pcx + 0.5 * pw, 0.0, IMG_W - 1.0)
    y2 = jnp.clip(pcy + 0.5 * ph, 0.0, IMG_H - 1.0)
    keep = ((x2 - x1 + 1.0 >= MIN_THRESHOLD)
            & (y2 - y1 + 1.0 >= MIN_THRESHOLD))
    x1o[...] = x1
    y1o[...] = y1
    x2o[...] = x2
    y2o[...] = y2
    mso[...] = jnp.where(keep, sc[...], NEG)
    kc = jnp.sum(keep.astype(jnp.float32), axis=(1, 2), keepdims=True)
    kco[...] = jnp.broadcast_to(kc, (B, 8, 128))


def _run_k1(dx, dy, dw, dh, sc):
    outs = pl.pallas_call(
        _k1_body,
        out_shape=(
            jax.ShapeDtypeStruct((B, ROWS, 128), jnp.float32),
            jax.ShapeDtypeStruct((B, ROWS, 128), jnp.float32),
            jax.ShapeDtypeStruct((B, ROWS, 128), jnp.float32),
            jax.ShapeDtypeStruct((B, ROWS, 128), jnp.float32),
            jax.ShapeDtypeStruct((B, ROWS, 128), jnp.float32),
            jax.ShapeDtypeStruct((B, 8, 128), jnp.float32),
        ),
    )(dx, dy, dw, dh, sc,
      jnp.asarray(_AW), jnp.asarray(_AH), jnp.asarray(_AX), jnp.asarray(_AY))
    return outs


_NB = 1024


def _k2a_body(msj, kcr, flago):
    bi = pl.program_id(0)
    kc = kcr[0, 0, 0].astype(jnp.int32)
    tgt = jnp.minimum(jnp.int32(M), kc)
    s = msj[0]
    k = jax.lax.bitcast_convert_type(s, jnp.int32)
    m = jnp.where(k >= 0, k, k ^ jnp.int32(0x7FFFFFFF))
    kel = s > (NEG * 0.5)
    b1 = (m >> 22) + _NB // 2
    b2 = (m >> 12) & (_NB - 1)
    bcol = lax.broadcasted_iota(jnp.int32, (_NB, 1), 0)
    cab = jnp.zeros((_NB, 1), jnp.float32)
    hb = jnp.zeros((_NB, 1), jnp.float32)
    for c in range(NP // _CJ):
        b1c = b1[:, c * _CJ:(c + 1) * _CJ]
        kc_ = kel[:, c * _CJ:(c + 1) * _CJ]
        cab = cab + jnp.sum(((b1c > bcol) & kc_).astype(jnp.float32),
                            axis=1, keepdims=True)
        hb = hb + jnp.sum(((b1c == bcol) & kc_).astype(jnp.float32),
                          axis=1, keepdims=True)
    tgtf = tgt.astype(jnp.float32)
    t1 = jnp.sum((cab >= tgtf).astype(jnp.int32))
    cab_t1 = jnp.sum(jnp.where(bcol == t1, cab, 0.0))
    t2g = tgtf - cab_t1
    cab2 = jnp.zeros((_NB, 1), jnp.float32)
    for c in range(NP // _CJ):
        b1c = b1[:, c * _CJ:(c + 1) * _CJ]
        b2c = b2[:, c * _CJ:(c + 1) * _CJ]
        kc_ = kel[:, c * _CJ:(c + 1) * _CJ]
        cab2 = cab2 + jnp.sum(((b1c == t1) & (b2c > bcol)
                               & kc_).astype(jnp.float32),
                              axis=1, keepdims=True)
    t2 = jnp.sum((cab2 >= t2g).astype(jnp.int32))
    flag = kel & ((b1 > t1) | ((b1 == t1) & (b2 >= t2)))
    flago[0] = flag.astype(jnp.float32)


def _run_k2a(ms, kc):
    msj = ms.reshape(B, 1, NP)
    flag = pl.pallas_call(
        _k2a_body,
        grid=(B,),
        in_specs=[
            pl.BlockSpec((1, 1, NP), lambda b: (b, 0, 0)),
            pl.BlockSpec((1, 8, 128), lambda b: (b, 0, 0)),
        ],
        out_specs=pl.BlockSpec((1, 1, NP), lambda b: (b, 0, 0)),
        out_shape=jax.ShapeDtypeStruct((B, 1, NP), jnp.float32),
    )(msj, kc)
    return flag.reshape(B, NP)


NCAP = 8192


def _k2c_body(flag_h, x1_h, y1_h, x2_h, y2_h, ss_h,
              c1_h, c2_h, c3_h, c4_h, c5_h,
              fv, i1, i2, i3, i4, i5, o1, o2, o3, o4, o5):
    wid = lax.axis_index("c") * _NS + lax.axis_index("s")

    @pl.when(wid < B)
    def _():
        bidx = wid
        neg = jnp.full((16,), -1.0e38, jnp.float32)
        zf = jnp.zeros((16,), jnp.float32)

        def init(k2, c):
            o16 = pl.multiple_of(k2 * 16, 16)
            o1[pl.ds(o16, 16)] = zf
            o2[pl.ds(o16, 16)] = zf
            o3[pl.ds(o16, 16)] = zf
            o4[pl.ds(o16, 16)] = zf
            o5[pl.ds(o16, 16)] = neg
            return c

        lax.fori_loop(0, NCAP // 16, init, 0)

        def macro(mc, base0):
            off = pl.multiple_of(mc * _MC, _MC)
            pltpu.sync_copy(flag_h.at[bidx, pl.ds(off, _MC)], fv)
            pltpu.sync_copy(x1_h.at[bidx, pl.ds(off, _MC)], i1)
            pltpu.sync_copy(y1_h.at[bidx, pl.ds(off, _MC)], i2)
            pltpu.sync_copy(x2_h.at[bidx, pl.ds(off, _MC)], i3)
            pltpu.sync_copy(y2_h.at[bidx, pl.ds(off, _MC)], i4)
            pltpu.sync_copy(ss_h.at[bidx, pl.ds(off, _MC)], i5)

            def chunk(k2, base):
                o16 = pl.multiple_of(k2 * 16, 16)
                kv = fv[pl.ds(o16, 16)] > 0.5
                ki = kv.astype(jnp.int32)
                slot = base + plsc.cumsum(ki) - 1
                m2 = kv & (slot < NCAP)
                slotc = jnp.where(m2, slot, 0)
                plsc.store_scatter(o1, [slotc], i1[pl.ds(o16, 16)], mask=m2)
                plsc.store_scatter(o2, [slotc], i2[pl.ds(o16, 16)], mask=m2)
                plsc.store_scatter(o3, [slotc], i3[pl.ds(o16, 16)], mask=m2)
                plsc.store_scatter(o4, [slotc], i4[pl.ds(o16, 16)], mask=m2)
                plsc.store_scatter(o5, [slotc], i5[pl.ds(o16, 16)], mask=m2)
                return base + jnp.sum(ki)

            return lax.fori_loop(0, _MC // 16, chunk, base0)

        lax.fori_loop(0, NP // _MC, macro, jnp.int32(0))
        pltpu.sync_copy(o1, c1_h.at[bidx])
        pltpu.sync_copy(o2, c2_h.at[bidx])
        pltpu.sync_copy(o3, c3_h.at[bidx])
        pltpu.sync_copy(o4, c4_h.at[bidx])
        pltpu.sync_copy(o5, c5_h.at[bidx])


def _run_k2c(flag, x1, y1, x2, y2, ms):
    fvec = jax.ShapeDtypeStruct((B, NCAP), jnp.float32)
    run = pl.kernel(
        _k2c_body,
        out_type=(fvec,) * 5,
        compiler_params=pltpu.CompilerParams(needs_layout_passes=False),
        mesh=plsc.VectorSubcoreMesh(core_axis_name="c", subcore_axis_name="s"),
        scratch_types=[pltpu.VMEM((_MC,), jnp.float32)] * 6
        + [pltpu.VMEM((NCAP,), jnp.float32)] * 5,
    )
    return run(flag, x1, y1, x2, y2, ms)


_TI = 512
_CJ = 2048


def _k2_body(msi, msj, ranko):
    t = pl.program_id(1)
    si = msi[0]
    ii = t * _TI + lax.broadcasted_iota(jnp.int32, (_TI, 1), 0)
    acc = jnp.zeros((_TI, 1), jnp.float32)
    for c in range(NCAP // _CJ):
        sj = msj[0, :, c * _CJ:(c + 1) * _CJ]
        jj = c * _CJ + lax.broadcasted_iota(jnp.int32, (1, _CJ), 1)
        gt = sj > si
        tie = (sj == si) & (jj < ii)
        acc = acc + jnp.sum((gt | tie).astype(jnp.float32), axis=1,
                            keepdims=True)
    ranko[0] = acc.astype(jnp.int32)


def _run_k2(cms):
    msi = cms.reshape(B, NCAP, 1)
    msj = cms.reshape(B, 1, NCAP)
    rank = pl.pallas_call(
        _k2_body,
        grid=(B, NCAP // _TI),
        in_specs=[
            pl.BlockSpec((1, _TI, 1), lambda b, t: (b, t, 0)),
            pl.BlockSpec((1, 1, NCAP), lambda b, t: (b, 0, 0)),
        ],
        out_specs=pl.BlockSpec((1, _TI, 1), lambda b, t: (b, t, 0)),
        out_shape=jax.ShapeDtypeStruct((B, NCAP, 1), jnp.int32),
    )(msi, msj)
    return rank.reshape(B, NCAP)


_MC = NP // 2
_NC = 2
_NS = 16


def _k3_body(rank_h, x1_h, y1_h, x2_h, y2_h, ss_h,
             o1_h, o2_h, o3_h, o4_h, o5_h,
             rank_v, i1, i2, i3, i4, i5, o1, o2, o3, o4, o5):
    wid = lax.axis_index("c") * _NS + lax.axis_index("s")

    @pl.when(wid < B)
    def _():
        bidx = wid
        pltpu.sync_copy(rank_h.at[bidx], rank_v)
        pltpu.sync_copy(x1_h.at[bidx], i1)
        pltpu.sync_copy(y1_h.at[bidx], i2)
        pltpu.sync_copy(x2_h.at[bidx], i3)
        pltpu.sync_copy(y2_h.at[bidx], i4)
        pltpu.sync_copy(ss_h.at[bidx], i5)

        def chunk(k, c2):
            o16 = pl.multiple_of(k * 16, 16)
            idx = rank_v[pl.ds(o16, 16)]
            msk = idx < M
            idxc = jnp.where(msk, idx, 0)
            plsc.store_scatter(o1, [idxc], i1[pl.ds(o16, 16)], mask=msk)
            plsc.store_scatter(o2, [idxc], i2[pl.ds(o16, 16)], mask=msk)
            plsc.store_scatter(o3, [idxc], i3[pl.ds(o16, 16)], mask=msk)
            plsc.store_scatter(o4, [idxc], i4[pl.ds(o16, 16)], mask=msk)
            plsc.store_scatter(o5, [idxc], i5[pl.ds(o16, 16)], mask=msk)
            return c2

        lax.fori_loop(0, NCAP // 16, chunk, 0)
        pltpu.sync_copy(o1, o1_h.at[bidx])
        pltpu.sync_copy(o2, o2_h.at[bidx])
        pltpu.sync_copy(o3, o3_h.at[bidx])
        pltpu.sync_copy(o4, o4_h.at[bidx])
        pltpu.sync_copy(o5, o5_h.at[bidx])


def _run_k3(rank, x1, y1, x2, y2, ms):
    fvec = jax.ShapeDtypeStruct((B, M), jnp.float32)
    run = pl.kernel(
        _k3_body,
        out_type=(fvec,) * 5,
        compiler_params=pltpu.CompilerParams(needs_layout_passes=False),
        mesh=plsc.VectorSubcoreMesh(core_axis_name="c", subcore_axis_name="s"),
        scratch_types=[
            pltpu.VMEM((NCAP,), jnp.int32),
            pltpu.VMEM((NCAP,), jnp.float32),
            pltpu.VMEM((NCAP,), jnp.float32),
            pltpu.VMEM((NCAP,), jnp.float32),
            pltpu.VMEM((NCAP,), jnp.float32),
            pltpu.VMEM((NCAP,), jnp.float32),
            pltpu.VMEM((M,), jnp.float32),
            pltpu.VMEM((M,), jnp.float32),
            pltpu.VMEM((M,), jnp.float32),
            pltpu.VMEM((M,), jnp.float32),
            pltpu.VMEM((M,), jnp.float32),
        ],
    )
    return run(rank, x1, y1, x2, y2, ms)


_MR = M // 128
_CHR = 2


def _k4_body(x1r, y1r, x2r, y2r, kcr, kepto):
    kc = jnp.min(kcr[...]).astype(jnp.int32)
    pre_n = jnp.minimum(PRE_NMS_TOPN, kc)
    sub = lax.broadcasted_iota(jnp.int32, (B, _MR, 128), 1)
    lan = lax.broadcasted_iota(jnp.int32, (B, _MR, 128), 2)
    pos = sub * 128 + lan
    active = (pos < pre_n).astype(jnp.float32)
    ploc = (lax.broadcasted_iota(jnp.int32, (B, _CHR, 128), 1) * 128
            + lax.broadcasted_iota(jnp.int32, (B, _CHR, 128), 2))
    kept_chunks = []
    cnt = jnp.zeros((B, 1, 1), jnp.float32)
    x1f, y1f, x2f, y2f = x1r[...], y1r[...], x2r[...], y2r[...]
    topn = jnp.float32(POST_NMS_TOPN)
    for k in range(_MR // _CHR):
        r0 = k * _CHR
        x1 = x1f[:, r0:, :]
        y1 = y1f[:, r0:, :]
        x2 = x2f[:, r0:, :]
        y2 = y2f[:, r0:, :]
        areas = (x2 - x1 + 1.0) * (y2 - y1 + 1.0)
        prel = pos[:, r0:, :] - r0 * 128
        xc1 = x1[:, :_CHR, :]
        yc1 = y1[:, :_CHR, :]
        xc2 = x2[:, :_CHR, :]
        yc2 = y2[:, :_CHR, :]

        def body(jj, st):
            act, keptc, cn = st
            mjf = (ploc == jj).astype(jnp.float32)
            actc = act[:, :_CHR, :]
            kb0 = jnp.sum(actc * mjf, axis=(1, 2), keepdims=True)
            kb = kb0 * (cn < topn).astype(jnp.float32)
            ex1 = jnp.sum(xc1 * mjf, axis=(1, 2), keepdims=True)
            ey1 = jnp.sum(yc1 * mjf, axis=(1, 2), keepdims=True)
            ex2 = jnp.sum(xc2 * mjf, axis=(1, 2), keepdims=True)
            ey2 = jnp.sum(yc2 * mjf, axis=(1, 2), keepdims=True)
            aj = (ex2 - ex1 + 1.0) * (ey2 - ey1 + 1.0)
            xx1 = jnp.maximum(ex1, x1)
            yy1 = jnp.maximum(ey1, y1)
            xx2 = jnp.minimum(ex2, x2)
            yy2 = jnp.minimum(ey2, y2)
            ww = jnp.maximum(0.0, xx2 - xx1 + 1.0)
            hh = jnp.maximum(0.0, yy2 - yy1 + 1.0)
            inter = ww * hh
            iou = inter / (aj + areas - inter)
            supf = ((kb > 0.0) & (iou > NMS_THRESH)
                    & (prel > jj)).astype(jnp.float32)
            keptc = jnp.maximum(keptc, kb * mjf)
            act = act * (1.0 - supf)
            return act, keptc, cn + kb

        def run_chunk(st):
            return lax.fori_loop(0, _CHR * 128, body, st)

        def skip_chunk(st):
            return st

        active, keptc, cnt = lax.cond(
            jnp.min(cnt) < topn, run_chunk, skip_chunk,
            (active, jnp.zeros((B, _CHR, 128), jnp.float32), cnt))
        kept_chunks.append(keptc)
        active = active[:, _CHR:, :]
    kepto[...] = jnp.concatenate(kept_chunks, axis=1)


def _run_k4(xs1, ys1, xs2, ys2, kc):
    shp = (B, _MR, 128)
    kept = pl.pallas_call(
        _k4_body,
        out_shape=jax.ShapeDtypeStruct(shp, jnp.float32),
    )(xs1.reshape(shp), ys1.reshape(shp), xs2.reshape(shp), ys2.reshape(shp),
      kc)
    return kept.reshape(B, M)


def _k5_body(kept_h, x1_h, y1_h, x2_h, y2_h, ss_h,
             ob_h, os_h,
             kept_v, i1, i2, i3, i4, i5, ob, os_):
    wid = lax.axis_index("c") * _NS + lax.axis_index("s")

    @pl.when(wid < B)
    def _():
        bidx = wid
        pltpu.sync_copy(kept_h.at[bidx], kept_v)
        pltpu.sync_copy(x1_h.at[bidx], i1)
        pltpu.sync_copy(y1_h.at[bidx], i2)
        pltpu.sync_copy(x2_h.at[bidx], i3)
        pltpu.sync_copy(y2_h.at[bidx], i4)
        pltpu.sync_copy(ss_h.at[bidx], i5)

        zf = jnp.zeros((16,), jnp.float32)

        def zb(k, c):
            ob[pl.ds(pl.multiple_of(k * 16, 16), 16)] = zf
            return c

        lax.fori_loop(0, (OUT_SLOTS * 4) // 16, zb, 0)

        def zs(k, c):
            os_[pl.ds(pl.multiple_of(k * 16, 16), 16)] = zf
            return c

        lax.fori_loop(0, OUT_SLOTS // 16, zs, 0)

        def chunk(k, base):
            o16 = pl.multiple_of(k * 16, 16)
            kv = kept_v[pl.ds(o16, 16)] > 0.5
            ki = kv.astype(jnp.int32)
            slot = base + plsc.cumsum(ki) - 1
            m2 = kv & (slot < POST_NMS_TOPN)
            slotc = jnp.where(m2, slot, 0)
            plsc.store_scatter(os_, [slotc], i5[pl.ds(o16, 16)], mask=m2)
            sb = slotc * 4
            plsc.store_scatter(ob, [sb], i1[pl.ds(o16, 16)], mask=m2)
            plsc.store_scatter(ob, [sb + 1], i2[pl.ds(o16, 16)], mask=m2)
            plsc.store_scatter(ob, [sb + 2], i3[pl.ds(o16, 16)], mask=m2)
            plsc.store_scatter(ob, [sb + 3], i4[pl.ds(o16, 16)], mask=m2)
            return base + jnp.sum(ki)

        lax.fori_loop(0, M // 16, chunk, jnp.int32(0))
        pltpu.sync_copy(ob, ob_h.at[bidx])
        pltpu.sync_copy(os_, os_h.at[bidx])


def _run_k5(kept, xs1, ys1, xs2, ys2, ss):
    run = pl.kernel(
        _k5_body,
        out_type=(
            jax.ShapeDtypeStruct((B, OUT_SLOTS * 4), jnp.float32),
            jax.ShapeDtypeStruct((B, OUT_SLOTS), jnp.float32),
        ),
        compiler_params=pltpu.CompilerParams(needs_layout_passes=False),
        mesh=plsc.VectorSubcoreMesh(core_axis_name="c", subcore_axis_name="s"),
        scratch_types=[
            pltpu.VMEM((M,), jnp.float32),
            pltpu.VMEM((M,), jnp.float32),
            pltpu.VMEM((M,), jnp.float32),
            pltpu.VMEM((M,), jnp.float32),
            pltpu.VMEM((M,), jnp.float32),
            pltpu.VMEM((M,), jnp.float32),
            pltpu.VMEM((OUT_SLOTS * 4,), jnp.float32),
            pltpu.VMEM((OUT_SLOTS,), jnp.float32),
        ],
    )
    return run(kept, xs1, ys1, xs2, ys2, ss)


@jax.jit
def kernel(labels_pred, bbox_reg):
    scores = jnp.transpose(labels_pred, (0, 2, 3, 1)).reshape(B, N, 2)[..., 1]
    breg = jnp.transpose(bbox_reg, (0, 2, 3, 1)).reshape(B, N, 4)
    pad = ((0, 0), (0, NP - N))

    def prep(v):
        return jnp.pad(v, pad).reshape(B, ROWS, 128)

    dx = prep(breg[..., 0])
    dy = prep(breg[..., 1])
    dw = prep(breg[..., 2])
    dh = prep(breg[..., 3])
    sc = prep(scores)

    x1, y1, x2, y2, ms, kc = _run_k1(dx, dy, dw, dh, sc)
    flat = lambda v: v.reshape(B, NP)
    flag = _run_k2a(flat(ms), kc)
    c1, c2, c3, c4, c5 = _run_k2c(flag, flat(x1), flat(y1), flat(x2),
                                  flat(y2), flat(ms))
    crank = _run_k2(c5)
    xs1, ys1, xs2, ys2, ss = _run_k3(crank, c1, c2, c3, c4, c5)
    kept = _run_k4(xs1, ys1, xs2, ys2, kc)
    ob, os_ = _run_k5(kept, xs1, ys1, xs2, ys2, ss)
    boxes_out = ob.reshape(B, OUT_SLOTS, 4)[:, :POST_NMS_TOPN]
    scores_out = os_[:, :POST_NMS_TOPN]
    return boxes_out, scores_out
